# baseline (speedup 1.0000x reference)
import functools

import jax
import jax.numpy as jnp
from jax import lax
from jax.experimental import pallas as pl
from jax.experimental.pallas import tpu as pltpu
from jax.experimental.pallas import tpu_sc as plsc

_ROWS = 4096
_COLS = 50
_EMBED = 64
_B = _ROWS * _COLS
_NW = 32
_BPW = _B // _NW
_CHUNK = 400
_NCHUNK = _BPW // _CHUNK
_NBUF = 4

_mesh = plsc.VectorSubcoreMesh(core_axis_name="c", subcore_axis_name="s")


@functools.partial(
    pl.kernel,
    mesh=_mesh,
    compiler_params=pltpu.CompilerParams(use_tc_tiling_on_sc=False),
    out_type=jax.ShapeDtypeStruct((_B, _EMBED), jnp.float32),
    scratch_types=[
        pltpu.VMEM((_BPW,), jnp.int32),
        pltpu.VMEM((_NBUF, _CHUNK, _EMBED), jnp.float32),
        pltpu.SemaphoreType.DMA((_NBUF,)),
        pltpu.SemaphoreType.DMA((_NBUF,)),
    ],
)
def _sc_gather(x_hbm, enc_hbm, out_hbm, idx_v, rows_v, gsem, ssem):
    wid = lax.axis_index("s") * 2 + lax.axis_index("c")
    base = wid * _BPW

    pltpu.sync_copy(x_hbm.at[pl.ds(base, _BPW)], idx_v)

    def gather(c):
        b = c % _NBUF
        return pltpu.async_copy(
            enc_hbm.at[idx_v.at[pl.ds(c * _CHUNK, _CHUNK)]],
            rows_v.at[b], gsem.at[b])

    def store(c):
        b = c % _NBUF
        return pltpu.async_copy(
            rows_v.at[b], out_hbm.at[pl.ds(base + c * _CHUNK, _CHUNK)],
            ssem.at[b])

    gathers = {0: gather(0), 1: gather(1)}
    stores = {}
    for i in range(_NCHUNK):
        c = i + 2
        if c < _NCHUNK:
            if c >= _NBUF:
                stores[c - _NBUF].wait()
            gathers[c] = gather(c)
        gathers[i].wait()
        stores[i] = store(i)
    for i in range(_NCHUNK - _NBUF, _NCHUNK):
        stores[i].wait()


def kernel(x, encoding):
    out = _sc_gather(x.reshape(_B), encoding)
    return out.reshape(_ROWS, _COLS, _EMBED)


# speedup vs baseline: 1.1296x; 1.1296x over previous
import functools

import jax
import jax.numpy as jnp
from jax import lax
from jax.experimental import pallas as pl
from jax.experimental.pallas import tpu as pltpu
from jax.experimental.pallas import tpu_sc as plsc

_ROWS = 4096
_COLS = 50
_EMBED = 64
_VOCAB = 1000
_NW = 32
_PPW = _ROWS // _NW
_PB = 2
_NCHUNK = _PPW // _PB
_NBUF = 2
_L = 16

_GROUPS = ((0, range(_L)), (16, range(_L)), (32, range(_L)), (34, (14, 15)))

_mesh = plsc.VectorSubcoreMesh(core_axis_name="c", subcore_axis_name="s")


@functools.partial(
    pl.kernel,
    mesh=_mesh,
    out_type=jax.ShapeDtypeStruct((_ROWS, _COLS, _EMBED), jnp.float32),
    scratch_types=[
        pltpu.VMEM((_VOCAB * _EMBED,), jnp.float32),
        pltpu.VMEM((_PPW, _COLS), jnp.int32),
        pltpu.VMEM((_NBUF, _PB, _COLS, _EMBED), jnp.float32),
        pltpu.SemaphoreType.DMA((_NBUF,)),
    ],
)
def _sc_lookup(x_hbm, enc_hbm, out_hbm, tbl_v, idx_v, stage_v, ssem):
    wid = lax.axis_index("s") * 2 + lax.axis_index("c")
    base = wid * _PPW

    pltpu.sync_copy(enc_hbm, tbl_v)
    pltpu.sync_copy(x_hbm.at[pl.ds(base, _PPW)], idx_v)

    def chunk_body(c, carry):
        b = c % _NBUF
        dst = out_hbm.at[pl.ds(base + c * _PB, _PB)]

        @pl.when(c >= _NBUF)
        def _():
            pltpu.make_async_copy(stage_v.at[b], dst, ssem.at[b]).wait()

        for sp in range(_PB):
            cp = c * _PB + sp
            for r0, lanes in _GROUPS:
                iv = idx_v[cp, pl.ds(r0, _L)]
                for j in lanes:
                    off = iv[j] * _EMBED
                    for k in range(_EMBED // _L):
                        stage_v[b, sp, r0 + j, pl.ds(k * _L, _L)] = (
                            tbl_v[pl.ds(off + k * _L, _L)])
        pltpu.async_copy(stage_v.at[b], dst, ssem.at[b])
        return carry

    lax.fori_loop(0, _NCHUNK, chunk_body, 0)
    for c in range(_NCHUNK - _NBUF, _NCHUNK):
        b = c % _NBUF
        pltpu.make_async_copy(
            stage_v.at[b], out_hbm.at[pl.ds(base + c * _PB, _PB)],
            ssem.at[b]).wait()


def kernel(x, encoding):
    out = _sc_lookup(x, encoding.reshape(_VOCAB * _EMBED))
    return out


# speedup vs baseline: 1.3121x; 1.1615x over previous
import functools

import jax
import jax.numpy as jnp
from jax import lax
from jax.experimental import pallas as pl
from jax.experimental.pallas import tpu as pltpu
from jax.experimental.pallas import tpu_sc as plsc

_ROWS = 4096
_COLS = 50
_EMBED = 64
_VOCAB = 1000
_NW = 32
_IW = _ROWS // _NW
_NBUF = 2
_L = 16

_mesh = plsc.VectorSubcoreMesh(core_axis_name="c", subcore_axis_name="s")


@functools.partial(
    pl.kernel,
    mesh=_mesh,
    compiler_params=pltpu.CompilerParams(needs_layout_passes=False),
    out_type=jax.ShapeDtypeStruct((_COLS, _EMBED, _ROWS), jnp.float32),
    scratch_types=[
        pltpu.VMEM((_EMBED * _VOCAB,), jnp.float32),
        pltpu.VMEM((_COLS, _IW), jnp.int32),
        pltpu.VMEM((_NBUF, _EMBED, _IW), jnp.float32),
        pltpu.SemaphoreType.DMA((_NBUF,)),
    ],
)
def _sc_lookup(xt_hbm, enc_hbm, out_hbm, tbl_v, idx_v, stage_v, ssem):
    wid = lax.axis_index("s") * 2 + lax.axis_index("c")
    i0 = wid * _IW

    pltpu.sync_copy(enc_hbm, tbl_v)
    pltpu.sync_copy(xt_hbm.at[:, pl.ds(i0, _IW)], idx_v)

    def plane(j, carry):
        b = j % _NBUF
        dst = out_hbm.at[j, :, pl.ds(i0, _IW)]

        @pl.when(j >= _NBUF)
        def _():
            pltpu.make_async_copy(stage_v.at[b], dst, ssem.at[b]).wait()

        ivecs = [idx_v[j, pl.ds(g * _L, _L)] for g in range(_IW // _L)]
        for e in range(_EMBED):
            for g in range(_IW // _L):
                vals = plsc.load_gather(tbl_v, [ivecs[g] + e * _VOCAB])
                stage_v[b, e, pl.ds(g * _L, _L)] = vals
        pltpu.async_copy(stage_v.at[b], dst, ssem.at[b])
        return carry

    lax.fori_loop(0, _COLS, plane, 0)
    for j in range(_COLS - _NBUF, _COLS):
        pltpu.make_async_copy(
            stage_v.at[j % _NBUF], out_hbm.at[j, :, pl.ds(i0, _IW)],
            ssem.at[j % _NBUF]).wait()


def kernel(x, encoding):
    xt = x.T
    enc_t = encoding.T.reshape(_EMBED * _VOCAB)
    out2 = _sc_lookup(xt, enc_t)
    return out2.transpose(2, 0, 1)
